# baseline (speedup 1.0000x reference)
import functools

import jax
import jax.numpy as jnp
from jax import lax
from jax.experimental import pallas as pl
from jax.experimental.pallas import tpu as pltpu
from jax.experimental.pallas import tpu_sc as plsc

E = 8
TOPK = 2
D = 1024
H = 2048
N = 2048
A = N * TOPK
BN = 128
NPAD = A + E * BN
NT = NPAD // BN
EPAD = 128
CB = 512

NC, NS = 2, 16
NW = NC * NS
APW = A // NW
TPW = N // NW
CH = 32


def _gate_body(x_ref, wg_ref, bg_ref, pos_ref, wf_ref, be_ref,
               oh_ref, cs_ref):
    logits = jnp.dot(x_ref[...], wg_ref[...],
                     preferred_element_type=jnp.float32) + bg_ref[...]
    m = jnp.max(logits, axis=-1, keepdims=True)
    ex = jnp.exp(logits - m)
    s = ex / jnp.sum(ex, axis=-1, keepdims=True)

    ii = lax.broadcasted_iota(jnp.int32, (N, E), 1)
    m1 = jnp.max(s, axis=-1, keepdims=True)
    idx1 = jnp.min(jnp.where(s == m1, ii, E), axis=-1, keepdims=True)
    s2 = jnp.where(ii == idx1, -jnp.inf, s)
    m2 = jnp.max(s2, axis=-1, keepdims=True)
    idx2 = jnp.min(jnp.where(s2 == m2, ii, E), axis=-1, keepdims=True)
    wf_ref[...] = jnp.concatenate([m1, m2], axis=0)

    ef = jnp.concatenate([idx1, idx2], axis=0)
    ep = lax.broadcasted_iota(jnp.int32, (A, EPAD), 1)
    oh_ref[...] = (ep == ef).astype(jnp.float32)

    tri = (lax.broadcasted_iota(jnp.int32, (CB, CB), 1)
           <= lax.broadcasted_iota(jnp.int32, (CB, CB), 0)).astype(jnp.float32)
    run = jnp.zeros((1, EPAD), jnp.float32)
    for b in range(A // CB):
        blk = oh_ref[b * CB:(b + 1) * CB, :]
        loc = jnp.dot(tri, blk, preferred_element_type=jnp.float32) + run
        cs_ref[b * CB:(b + 1) * CB, :] = loc
        run = loc[CB - 1:CB, :]
    counts = run

    pc = jnp.floor((counts + (BN - 1)) * (1.0 / BN)) * BN
    su = (lax.broadcasted_iota(jnp.int32, (EPAD, EPAD), 0)
          < lax.broadcasted_iota(jnp.int32, (EPAD, EPAD), 1)).astype(jnp.float32)
    poff = jnp.dot(pc, su, preferred_element_type=jnp.float32)
    pend = poff + pc

    pos_f = jnp.sum(oh_ref[...] * (poff + cs_ref[...]), axis=-1,
                    keepdims=True) - 1.0
    pos_ref[...] = pos_f.astype(jnp.int32)

    tb = lax.broadcasted_iota(jnp.int32, (NT, EPAD), 0).astype(jnp.float32) * float(BN)
    be = jnp.sum((tb >= pend).astype(jnp.float32), axis=-1, keepdims=True)
    be_ref[...] = jnp.minimum(be, float(E - 1)).astype(jnp.int32)


def _gate_call(x, Wg, bg2):
    return pl.pallas_call(
        _gate_body,
        grid=(1,),
        in_specs=[
            pl.BlockSpec((N, D), lambda i: (0, 0)),
            pl.BlockSpec((D, E), lambda i: (0, 0)),
            pl.BlockSpec((1, E), lambda i: (0, 0)),
        ],
        out_specs=[
            pl.BlockSpec((A, 1), lambda i: (0, 0)),
            pl.BlockSpec((A, 1), lambda i: (0, 0)),
            pl.BlockSpec((NT, 1), lambda i: (0, 0)),
        ],
        out_shape=[
            jax.ShapeDtypeStruct((A, 1), jnp.int32),
            jax.ShapeDtypeStruct((A, 1), jnp.float32),
            jax.ShapeDtypeStruct((NT, 1), jnp.int32),
        ],
        scratch_shapes=[
            pltpu.VMEM((A, EPAD), jnp.float32),
            pltpu.VMEM((A, EPAD), jnp.float32),
        ],
        compiler_params=pltpu.CompilerParams(
            dimension_semantics=("arbitrary",),
        ),
    )(x, Wg, bg2)


def _sc_worker_id():
    return lax.axis_index("s") * NC + lax.axis_index("c")


def _sc_scatter_body(pos2_hbm, wf2_hbm, x_hbm, xs_hbm, ws_hbm, posb_v, wb_v,
                     rows_v, seml0, seml1, sems0, sems1):
    nch = APW // CH
    wid = _sc_worker_id()
    base = wid * APW
    t0 = lax.rem(base, N)
    pltpu.sync_copy(pos2_hbm.at[pl.ds(wid * nch, nch)], posb_v)
    pltpu.sync_copy(wf2_hbm.at[pl.ds(wid * nch, nch)], wb_v)
    seml = (seml0, seml1)
    sems = (sems0, sems1)
    loads = {
        0: pltpu.async_copy(x_hbm.at[pl.ds(t0, CH)], rows_v.at[0], seml0),
        1: pltpu.async_copy(x_hbm.at[pl.ds(t0 + CH, CH)], rows_v.at[1], seml1),
    }
    for ch in range(nch):
        b = ch % 2
        loads[ch].wait()
        scat = pltpu.async_copy(rows_v.at[b], xs_hbm.at[posb_v.at[ch]],
                                sems[b])
        pltpu.async_copy(wb_v.at[ch], ws_hbm.at[posb_v.at[ch]],
                         sems[b]).wait()
        scat.wait()
        if ch + 2 < nch:
            loads[ch + 2] = pltpu.async_copy(
                x_hbm.at[pl.ds(t0 + (ch + 2) * CH, CH)], rows_v.at[b], seml[b])


def _sc_scatter(pos2, wf2, x):
    f = pl.kernel(
        _sc_scatter_body,
        out_type=(jax.ShapeDtypeStruct((NPAD, D), jnp.float32),
                  jax.ShapeDtypeStruct((NPAD,), jnp.float32)),
        mesh=plsc.VectorSubcoreMesh(core_axis_name="c", subcore_axis_name="s",
                                    num_cores=NC, num_subcores=NS),
        scratch_types=[
            pltpu.VMEM((APW // CH, CH), jnp.int32),
            pltpu.VMEM((APW // CH, CH), jnp.float32),
            pltpu.VMEM((2, CH, D), jnp.float32),
            pltpu.SemaphoreType.DMA,
            pltpu.SemaphoreType.DMA,
            pltpu.SemaphoreType.DMA,
            pltpu.SemaphoreType.DMA,
        ],
    )
    return f(pos2, wf2, x)


def _sc_combine_body(p02_hbm, p12_hbm, ys_hbm, out_hbm,
                     pb0_v, pb1_v, rows_v, sema, semb):
    nch = TPW // CH
    wid = _sc_worker_id()
    base = wid * TPW
    pltpu.sync_copy(p02_hbm.at[pl.ds(wid * nch, nch)], pb0_v)
    pltpu.sync_copy(p12_hbm.at[pl.ds(wid * nch, nch)], pb1_v)
    ga = pltpu.async_copy(ys_hbm.at[pb0_v.at[0]], rows_v.at[0], sema)
    gb = pltpu.async_copy(ys_hbm.at[pb1_v.at[0]], rows_v.at[1], semb)
    for ch in range(nch):
        t0 = base + ch * CH
        ga.wait()
        gb.wait()

        def add_row(r, carry):
            for c in range(D // 16):
                sl = pl.ds(16 * c, 16)
                rows_v[0, r, sl] = rows_v[0, r, sl] + rows_v[1, r, sl]
            return carry

        lax.fori_loop(0, CH, add_row, 0, unroll=False)
        wa = pltpu.async_copy(rows_v.at[0], out_hbm.at[pl.ds(t0, CH)], sema)
        if ch + 1 < nch:
            gb = pltpu.async_copy(ys_hbm.at[pb1_v.at[ch + 1]], rows_v.at[1],
                                  semb)
        wa.wait()
        if ch + 1 < nch:
            ga = pltpu.async_copy(ys_hbm.at[pb0_v.at[ch + 1]], rows_v.at[0],
                                  sema)


def _sc_combine(p02, p12, ys):
    f = pl.kernel(
        _sc_combine_body,
        out_type=jax.ShapeDtypeStruct((N, D), jnp.float32),
        mesh=plsc.VectorSubcoreMesh(core_axis_name="c", subcore_axis_name="s",
                                    num_cores=NC, num_subcores=NS),
        scratch_types=[
            pltpu.VMEM((TPW // CH, CH), jnp.int32),
            pltpu.VMEM((TPW // CH, CH), jnp.int32),
            pltpu.VMEM((2, CH, D), jnp.float32),
            pltpu.SemaphoreType.DMA,
            pltpu.SemaphoreType.DMA,
        ],
    )
    return f(p02, p12, ys)


def _ffn_body(be_ref, xs_ref, ws_ref, w1_ref, b1_ref, w2_ref, b2_ref, ys_ref):
    h = jnp.dot(xs_ref[...], w1_ref[0], preferred_element_type=jnp.float32)
    h = h + b1_ref[0]
    h = 0.5 * h * (1.0 + lax.erf(h * 0.7071067811865476))
    acc = jnp.dot(h, w2_ref[0], preferred_element_type=jnp.float32) + b2_ref[0]
    ys_ref[...] = ws_ref[...] * acc


def _ffn_call(be, xs, ws2, W1, b1r, W2, b2r):
    grid_spec = pltpu.PrefetchScalarGridSpec(
        num_scalar_prefetch=1,
        grid=(NT,),
        in_specs=[
            pl.BlockSpec((BN, D), lambda t, be: (t, 0)),
            pl.BlockSpec((BN, 1), lambda t, be: (t, 0)),
            pl.BlockSpec((1, D, H), lambda t, be: (be[t], 0, 0)),
            pl.BlockSpec((1, 1, H), lambda t, be: (be[t], 0, 0)),
            pl.BlockSpec((1, H, D), lambda t, be: (be[t], 0, 0)),
            pl.BlockSpec((1, 1, D), lambda t, be: (be[t], 0, 0)),
        ],
        out_specs=pl.BlockSpec((BN, D), lambda t, be: (t, 0)),
    )
    return pl.pallas_call(
        _ffn_body,
        grid_spec=grid_spec,
        out_shape=jax.ShapeDtypeStruct((NPAD, D), jnp.float32),
        compiler_params=pltpu.CompilerParams(
            dimension_semantics=("arbitrary",),
        ),
    )(be, xs, ws2, W1, b1r, W2, b2r)


@jax.jit
def kernel(x, Wg, bg, W1, b1, W2, b2):
    bg2 = bg.reshape(1, E)
    b1r = b1.reshape(E, 1, H)
    b2r = b2.reshape(E, 1, D)
    pos, wf, be = _gate_call(x, Wg, bg2)
    pos2 = pos.reshape(A // CH, CH)
    xs, ws = _sc_scatter(pos2, wf.reshape(A // CH, CH), x)
    ys = _ffn_call(be.reshape(NT), xs, ws.reshape(NPAD, 1), W1, b1r, W2, b2r)
    return _sc_combine(pos2[:N // CH], pos2[N // CH:], ys)


# speedup vs baseline: 1.0986x; 1.0986x over previous
import functools

import jax
import jax.numpy as jnp
from jax import lax
from jax.experimental import pallas as pl
from jax.experimental.pallas import tpu as pltpu
from jax.experimental.pallas import tpu_sc as plsc

E = 8
TOPK = 2
D = 1024
H = 2048
N = 2048
A = N * TOPK
BN = 128
NPAD = A + E * BN
NT = NPAD // BN
EPAD = 128
CB = 512

NC, NS = 2, 16
NW = NC * NS
APW = A // NW
TPW = N // NW
CH = 32


def _gate_body(x_ref, wg_ref, bg_ref, pos_ref, w0_ref, w1_ref, be_ref,
               oh_ref, cs_ref):
    logits = jnp.dot(x_ref[...], wg_ref[...],
                     preferred_element_type=jnp.float32) + bg_ref[...]
    m = jnp.max(logits, axis=-1, keepdims=True)
    ex = jnp.exp(logits - m)
    s = ex / jnp.sum(ex, axis=-1, keepdims=True)

    ii = lax.broadcasted_iota(jnp.int32, (N, E), 1)
    m1 = jnp.max(s, axis=-1, keepdims=True)
    idx1 = jnp.min(jnp.where(s == m1, ii, E), axis=-1, keepdims=True)
    s2 = jnp.where(ii == idx1, -jnp.inf, s)
    m2 = jnp.max(s2, axis=-1, keepdims=True)
    idx2 = jnp.min(jnp.where(s2 == m2, ii, E), axis=-1, keepdims=True)
    w0_ref[...] = m1
    w1_ref[...] = m2

    ef = jnp.concatenate([idx1, idx2], axis=0)
    ep = lax.broadcasted_iota(jnp.int32, (A, EPAD), 1)
    oh_ref[...] = (ep == ef).astype(jnp.float32)

    tri = (lax.broadcasted_iota(jnp.int32, (CB, CB), 1)
           <= lax.broadcasted_iota(jnp.int32, (CB, CB), 0)).astype(jnp.float32)
    run = jnp.zeros((1, EPAD), jnp.float32)
    for b in range(A // CB):
        blk = oh_ref[b * CB:(b + 1) * CB, :]
        loc = jnp.dot(tri, blk, preferred_element_type=jnp.float32) + run
        cs_ref[b * CB:(b + 1) * CB, :] = loc
        run = loc[CB - 1:CB, :]
    counts = run

    pc = jnp.floor((counts + (BN - 1)) * (1.0 / BN)) * BN
    su = (lax.broadcasted_iota(jnp.int32, (EPAD, EPAD), 0)
          < lax.broadcasted_iota(jnp.int32, (EPAD, EPAD), 1)).astype(jnp.float32)
    poff = jnp.dot(pc, su, preferred_element_type=jnp.float32)
    pend = poff + pc

    pos_f = jnp.sum(oh_ref[...] * (poff + cs_ref[...]), axis=-1,
                    keepdims=True) - 1.0
    pos_ref[...] = pos_f.astype(jnp.int32)

    tb = lax.broadcasted_iota(jnp.int32, (NT, EPAD), 0).astype(jnp.float32) * float(BN)
    be = jnp.sum((tb >= pend).astype(jnp.float32), axis=-1, keepdims=True)
    be_ref[...] = jnp.minimum(be, float(E - 1)).astype(jnp.int32)


def _gate_call(x, Wg, bg2):
    return pl.pallas_call(
        _gate_body,
        grid=(1,),
        in_specs=[
            pl.BlockSpec((N, D), lambda i: (0, 0)),
            pl.BlockSpec((D, E), lambda i: (0, 0)),
            pl.BlockSpec((1, E), lambda i: (0, 0)),
        ],
        out_specs=[
            pl.BlockSpec((A, 1), lambda i: (0, 0)),
            pl.BlockSpec((N, 1), lambda i: (0, 0)),
            pl.BlockSpec((N, 1), lambda i: (0, 0)),
            pl.BlockSpec((NT, 1), lambda i: (0, 0)),
        ],
        out_shape=[
            jax.ShapeDtypeStruct((A, 1), jnp.int32),
            jax.ShapeDtypeStruct((N, 1), jnp.float32),
            jax.ShapeDtypeStruct((N, 1), jnp.float32),
            jax.ShapeDtypeStruct((NT, 1), jnp.int32),
        ],
        scratch_shapes=[
            pltpu.VMEM((A, EPAD), jnp.float32),
            pltpu.VMEM((A, EPAD), jnp.float32),
        ],
        compiler_params=pltpu.CompilerParams(
            dimension_semantics=("arbitrary",),
        ),
    )(x, Wg, bg2)


def _sc_worker_id():
    return lax.axis_index("s") * NC + lax.axis_index("c")


def _sc_scatter_body(pos2_hbm, x_hbm, xs_hbm, posb_v, rows_v,
                     seml0, seml1, sems0, sems1):
    nch = APW // CH
    wid = _sc_worker_id()
    base = wid * APW
    t0 = lax.rem(base, N)
    pltpu.sync_copy(pos2_hbm.at[pl.ds(wid * nch, nch)], posb_v)
    seml = (seml0, seml1)
    sems = (sems0, sems1)
    loads = {
        0: pltpu.async_copy(x_hbm.at[pl.ds(t0, CH)], rows_v.at[0], seml0),
        1: pltpu.async_copy(x_hbm.at[pl.ds(t0 + CH, CH)], rows_v.at[1], seml1),
    }
    for ch in range(nch):
        b = ch % 2
        loads[ch].wait()
        scat = pltpu.async_copy(rows_v.at[b], xs_hbm.at[posb_v.at[ch]],
                                sems[b])
        scat.wait()
        if ch + 2 < nch:
            loads[ch + 2] = pltpu.async_copy(
                x_hbm.at[pl.ds(t0 + (ch + 2) * CH, CH)], rows_v.at[b], seml[b])


def _sc_scatter(pos2, x):
    f = pl.kernel(
        _sc_scatter_body,
        out_type=jax.ShapeDtypeStruct((NPAD, D), jnp.float32),
        mesh=plsc.VectorSubcoreMesh(core_axis_name="c", subcore_axis_name="s",
                                    num_cores=NC, num_subcores=NS),
        scratch_types=[
            pltpu.VMEM((APW // CH, CH), jnp.int32),
            pltpu.VMEM((2, CH, D), jnp.float32),
            pltpu.SemaphoreType.DMA,
            pltpu.SemaphoreType.DMA,
            pltpu.SemaphoreType.DMA,
            pltpu.SemaphoreType.DMA,
        ],
    )
    return f(pos2, x)


def _sc_gather_body(p02_hbm, p12_hbm, ys_hbm, g0_hbm, g1_hbm,
                    pb0_v, pb1_v, rows_v, sema, semb):
    nch = TPW // CH
    wid = _sc_worker_id()
    base = wid * TPW
    pltpu.sync_copy(p02_hbm.at[pl.ds(wid * nch, nch)], pb0_v)
    pltpu.sync_copy(p12_hbm.at[pl.ds(wid * nch, nch)], pb1_v)
    ga = pltpu.async_copy(ys_hbm.at[pb0_v.at[0]], rows_v.at[0], sema)
    gb = pltpu.async_copy(ys_hbm.at[pb1_v.at[0]], rows_v.at[1], semb)
    for ch in range(nch):
        t0 = base + ch * CH
        ga.wait()
        wa = pltpu.async_copy(rows_v.at[0], g0_hbm.at[pl.ds(t0, CH)], sema)
        gb.wait()
        wb = pltpu.async_copy(rows_v.at[1], g1_hbm.at[pl.ds(t0, CH)], semb)
        wa.wait()
        if ch + 1 < nch:
            ga = pltpu.async_copy(ys_hbm.at[pb0_v.at[ch + 1]], rows_v.at[0],
                                  sema)
        wb.wait()
        if ch + 1 < nch:
            gb = pltpu.async_copy(ys_hbm.at[pb1_v.at[ch + 1]], rows_v.at[1],
                                  semb)


def _sc_gather(p02, p12, ys):
    f = pl.kernel(
        _sc_gather_body,
        out_type=(jax.ShapeDtypeStruct((N, D), jnp.float32),
                  jax.ShapeDtypeStruct((N, D), jnp.float32)),
        mesh=plsc.VectorSubcoreMesh(core_axis_name="c", subcore_axis_name="s",
                                    num_cores=NC, num_subcores=NS),
        scratch_types=[
            pltpu.VMEM((TPW // CH, CH), jnp.int32),
            pltpu.VMEM((TPW // CH, CH), jnp.int32),
            pltpu.VMEM((2, CH, D), jnp.float32),
            pltpu.SemaphoreType.DMA,
            pltpu.SemaphoreType.DMA,
        ],
    )
    return f(p02, p12, ys)


def _ffn_body(be_ref, xs_ref, w1_ref, b1_ref, w2_ref, b2_ref, ys_ref):
    h = jnp.dot(xs_ref[...], w1_ref[0], preferred_element_type=jnp.float32)
    h = h + b1_ref[0]
    h = 0.5 * h * (1.0 + lax.erf(h * 0.7071067811865476))
    ys_ref[...] = jnp.dot(h, w2_ref[0],
                          preferred_element_type=jnp.float32) + b2_ref[0]


def _ffn_call(be, xs, W1, b1r, W2, b2r):
    grid_spec = pltpu.PrefetchScalarGridSpec(
        num_scalar_prefetch=1,
        grid=(NT,),
        in_specs=[
            pl.BlockSpec((BN, D), lambda t, be: (t, 0)),
            pl.BlockSpec((1, D, H), lambda t, be: (be[t], 0, 0)),
            pl.BlockSpec((1, 1, H), lambda t, be: (be[t], 0, 0)),
            pl.BlockSpec((1, H, D), lambda t, be: (be[t], 0, 0)),
            pl.BlockSpec((1, 1, D), lambda t, be: (be[t], 0, 0)),
        ],
        out_specs=pl.BlockSpec((BN, D), lambda t, be: (t, 0)),
    )
    return pl.pallas_call(
        _ffn_body,
        grid_spec=grid_spec,
        out_shape=jax.ShapeDtypeStruct((NPAD, D), jnp.float32),
        compiler_params=pltpu.CompilerParams(
            dimension_semantics=("arbitrary",),
        ),
    )(be, xs, W1, b1r, W2, b2r)


def _combine_body(w0_ref, w1_ref, g0_ref, g1_ref, out_ref):
    out_ref[...] = w0_ref[...] * g0_ref[...] + w1_ref[...] * g1_ref[...]


def _combine_call(w0, w1, g0, g1):
    blk = 256
    return pl.pallas_call(
        _combine_body,
        grid=(N // blk,),
        in_specs=[
            pl.BlockSpec((blk, 1), lambda i: (i, 0)),
            pl.BlockSpec((blk, 1), lambda i: (i, 0)),
            pl.BlockSpec((blk, D), lambda i: (i, 0)),
            pl.BlockSpec((blk, D), lambda i: (i, 0)),
        ],
        out_specs=pl.BlockSpec((blk, D), lambda i: (i, 0)),
        out_shape=jax.ShapeDtypeStruct((N, D), jnp.float32),
    )(w0, w1, g0, g1)


@jax.jit
def kernel(x, Wg, bg, W1, b1, W2, b2):
    bg2 = bg.reshape(1, E)
    b1r = b1.reshape(E, 1, H)
    b2r = b2.reshape(E, 1, D)
    pos, w0, w1, be = _gate_call(x, Wg, bg2)
    pos2 = pos.reshape(A // CH, CH)
    xs = _sc_scatter(pos2, x)
    ys = _ffn_call(be.reshape(NT), xs, W1, b1r, W2, b2r)
    g0, g1 = _sc_gather(pos2[:N // CH], pos2[N // CH:], ys)
    return _combine_call(w0, w1, g0, g1)
